# speedup vs baseline: 4.5912x; 4.5912x over previous
import functools

import jax
import jax.numpy as jnp
from jax import lax
from jax.experimental import pallas as pl
from jax.experimental.pallas import tpu as pltpu
from jax.experimental.pallas import tpu_sc as plsc

N = 10000
E = 320000
DIM = 128
MAX_DEG = 64

NC = 2
NS = 16
NW = NC * NS
CH = 80
EPW = E // NW
N_PAD = 10240
RPT = N_PAD // NS


def _sc_scatter_rows(x, src, dst):
  mesh = plsc.VectorSubcoreMesh(
      core_axis_name="c", subcore_axis_name="s", num_cores=NC, num_subcores=NS)

  @functools.partial(
      pl.kernel,
      out_type=jax.ShapeDtypeStruct((NC, N_PAD, DIM), jnp.float32),
      mesh=mesh,
      scratch_types=[
          pltpu.VMEM((CH,), jnp.int32),
          pltpu.VMEM((CH,), jnp.int32),
          pltpu.VMEM((CH, DIM), jnp.float32),
          pltpu.VMEM_SHARED((N_PAD, DIM), jnp.float32),
          pltpu.SemaphoreType.DMA,
      ],
  )
  def body(x_hbm, src_hbm, dst_hbm, out_hbm, src_v, dst_v, rows_v, acc_sh, sem):
    c = lax.axis_index("c")
    s = lax.axis_index("s")
    wid = s * NC + c

    def zrow(r, carry):
      for j in range(DIM // 16):
        rows_v[r, pl.ds(j * 16, 16)] = jnp.zeros((16,), jnp.float32)
      return carry

    lax.fori_loop(0, CH, zrow, 0)
    for k in range(RPT // CH):
      pltpu.sync_copy(rows_v, acc_sh.at[pl.ds(s * RPT + k * CH, CH)])
    plsc.subcore_barrier()

    def step(i, carry):
      base = pl.multiple_of(wid * EPW + i * CH, 8)
      pltpu.sync_copy(src_hbm.at[pl.ds(base, CH)], src_v)
      pltpu.sync_copy(dst_hbm.at[pl.ds(base, CH)], dst_v)
      pltpu.async_copy(x_hbm.at[src_v], rows_v, sem).wait()
      pltpu.sync_copy(rows_v, acc_sh.at[dst_v], add=True)
      return carry

    lax.fori_loop(0, EPW // CH, step, 0)
    plsc.subcore_barrier()

    for k in range(RPT // CH):
      r0 = s * RPT + k * CH
      pltpu.sync_copy(acc_sh.at[pl.ds(r0, CH)], rows_v)
      pltpu.sync_copy(rows_v, out_hbm.at[c, pl.ds(r0, CH)])

  return body(x, src, dst)


def _lrelu(x):
  return jnp.where(x >= 0, x, 0.01 * x)


def _bn(x, g, b):
  m = jnp.mean(x, axis=0, keepdims=True)
  v = jnp.mean((x - m) ** 2, axis=0, keepdims=True)
  return (x - m) / jnp.sqrt(v + 1e-5) * g + b


def _tc_z0(deg2, embed_table):
  def body(deg_ref, emb_ref, out_ref):
    iot = lax.broadcasted_iota(jnp.int32, (1, MAX_DEG), 1)
    oh = (deg_ref[:] == iot).astype(jnp.float32)
    out_ref[:] = jnp.dot(oh, emb_ref[:], preferred_element_type=jnp.float32)

  return pl.pallas_call(
      body, out_shape=jax.ShapeDtypeStruct((N, DIM), jnp.float32))(
          deg2, embed_table)


def _tc_gin(z, aggp, eps, W, b, g, be):
  def body(z_ref, aggp_ref, eps_ref, w_ref, b_ref, g_ref, be_ref, out_ref):
    agg = aggp_ref[0, :N, :] + aggp_ref[1, :N, :]
    pre = jnp.dot((1.0 + eps_ref[0, 0]) * z_ref[:] + agg, w_ref[:],
                  preferred_element_type=jnp.float32) + b_ref[:]
    out_ref[:] = _lrelu(_bn(pre, g_ref[:], be_ref[:]))

  return pl.pallas_call(
      body, out_shape=jax.ShapeDtypeStruct((N, DIM), jnp.float32))(
          z, aggp, eps, W, b, g, be)


def _tc_head(z0, h1, aggp, eps2, W2, b2, g2, be2, Wf1, bf1, gf, bef, Wf2, bf2):
  def body(z0_ref, h1_ref, aggp_ref, eps_ref, w2_ref, b2_ref, g2_ref, be2_ref,
           wf1_ref, bf1_ref, gf_ref, bef_ref, wf2_ref, bf2_ref, out_ref):
    agg = aggp_ref[0, :N, :] + aggp_ref[1, :N, :]
    pre2 = jnp.dot((1.0 + eps_ref[0, 0]) * h1_ref[:] + agg, w2_ref[:],
                   preferred_element_type=jnp.float32) + b2_ref[:]
    h2 = _lrelu(_bn(pre2, g2_ref[:], be2_ref[:]))
    xw = (jnp.dot(z0_ref[:], wf1_ref[0:DIM, :],
                  preferred_element_type=jnp.float32)
          + jnp.dot(h1_ref[:], wf1_ref[DIM:2 * DIM, :],
                    preferred_element_type=jnp.float32)
          + jnp.dot(h2, wf1_ref[2 * DIM:3 * DIM, :],
                    preferred_element_type=jnp.float32)
          + bf1_ref[:])
    h = _lrelu(_bn(xw, gf_ref[:], bef_ref[:]))
    o = jnp.dot(h, wf2_ref[:], preferred_element_type=jnp.float32) + bf2_ref[:]
    out_ref[:] = jax.nn.sigmoid(o)

  return pl.pallas_call(
      body, out_shape=jax.ShapeDtypeStruct((N, 1), jnp.float32))(
          z0, h1, aggp, eps2, W2, b2, g2, be2, Wf1, bf1, gf, bef, Wf2, bf2)


def kernel(node_deg, edge_index, embed_table, eps1, W1, b1, g1, be1, eps2, W2,
           b2, g2, be2, Wf1, bf1, gf, bef, Wf2, bf2):
  src = edge_index[0].astype(jnp.int32)
  dst = edge_index[1].astype(jnp.int32)
  deg2 = node_deg.astype(jnp.int32).reshape(N, 1)
  r1 = lambda a: a.reshape(1, -1)

  z0 = _tc_z0(deg2, embed_table)
  agg1p = _sc_scatter_rows(z0, src, dst)
  h1 = _tc_gin(z0, agg1p, eps1.reshape(1, 1), W1, r1(b1), r1(g1), r1(be1))
  agg2p = _sc_scatter_rows(h1, src, dst)
  out = _tc_head(z0, h1, agg2p, eps2.reshape(1, 1), W2, r1(b2), r1(g2),
                 r1(be2), Wf1, r1(bf1), r1(gf), r1(bef), Wf2, r1(bf2))
  return out[:, 0]
